# baseline (speedup 1.0000x reference)
import functools

import jax
import jax.numpy as jnp
from jax import lax
from jax.experimental import pallas as pl
from jax.experimental.pallas import tpu as pltpu
from jax.experimental.pallas import tpu_sc as plsc

N = 10000
D = 128
H1 = 64
H2 = 32
E = 320000

NPAD = 10240
ROWS_PER_TILE = NPAD // 16
EPW = E // 32
CHUNK = 128
CPT = 78
TAIL = EPW - CPT * CHUNK
DEGW = 4


def _sc_mesh():
    return plsc.VectorSubcoreMesh(core_axis_name="c", subcore_axis_name="s")


def _idx(ref, j):
    return ref.at[pl.ds(j * CHUNK, CHUNK)]


def _zero_rows(buf, nrows, ncols):
    zv = jnp.zeros((16,), jnp.float32)

    def body(i, _):
        for j in range(ncols // 16):
            buf[i, pl.ds(j * 16, 16)] = zv
        return 0

    lax.fori_loop(0, nrows, body, 0)


def _deg_kernel(dst1d, zeros8, ones8, out, dst_v, ones_v, acc, *ss):
    c = lax.axis_index("c")
    s = lax.axis_index("s")
    wid = c * 16 + s

    pltpu.sync_copy(ones8, ones_v)
    pltpu.sync_copy(zeros8, acc.at[pl.ds(s * ROWS_PER_TILE, ROWS_PER_TILE)])
    pltpu.sync_copy(dst1d.at[pl.ds(wid * EPW, EPW)], dst_v)
    plsc.subcore_barrier()

    @pl.loop(0, CPT, step=6)
    def _(j0):
        for k in range(6):
            j = j0 + k

            @pl.when(j >= 6)
            def _():
                pltpu.make_async_copy(
                    ones_v, acc.at[_idx(dst_v, j - 6)], ss[k]).wait()

            pltpu.async_copy(ones_v, acc.at[_idx(dst_v, j)], ss[k], add=True)

    pltpu.sync_copy(ones_v.at[pl.ds(0, TAIL)],
                    acc.at[dst_v.at[pl.ds(CPT * CHUNK, TAIL)]], add=True)

    for j in range(CPT - 6, CPT):
        pltpu.make_async_copy(
            ones_v, acc.at[_idx(dst_v, j)], ss[j % 6]).wait()

    plsc.subcore_barrier()
    pltpu.sync_copy(
        acc.at[pl.ds(s * ROWS_PER_TILE, ROWS_PER_TILE)],
        out.at[c, pl.ds(s * ROWS_PER_TILE, ROWS_PER_TILE)],
    )


NBUF = 6
LOOKAHEAD = 4


def _agg_body(H, table, src1d, dst1d, out, src_v, dst_v, tbuf, *rest):
    bufs = rest[:NBUF]
    acc = rest[NBUF]
    gs = rest[NBUF + 1:NBUF + 1 + NBUF]
    ss = rest[NBUF + 1 + NBUF:]
    c = lax.axis_index("c")
    s = lax.axis_index("s")
    wid = c * 16 + s
    row0 = s * ROWS_PER_TILE

    _zero_rows(bufs[0], CHUNK, H)
    for r in range(ROWS_PER_TILE // CHUNK):
        pltpu.sync_copy(bufs[0], acc.at[pl.ds(row0 + r * CHUNK, CHUNK)])

    pltpu.sync_copy(src1d.at[pl.ds(wid * EPW, EPW)], src_v)
    pltpu.sync_copy(dst1d.at[pl.ds(wid * EPW, EPW)], dst_v)

    for k in range(LOOKAHEAD):
        pltpu.async_copy(table.at[_idx(src_v, k)], bufs[k], gs[k])
    plsc.subcore_barrier()

    @pl.loop(0, CPT, step=NBUF)
    def _(j0):
        for k in range(NBUF):
            j = j0 + k
            bw = (k + NBUF - 2) % NBUF

            @pl.when(j >= 2)
            def _():
                pltpu.make_async_copy(
                    bufs[bw], acc.at[_idx(dst_v, j - 2)], ss[bw]).wait()

            @pl.when(j + LOOKAHEAD < CPT)
            def _():
                bg = (k + LOOKAHEAD) % NBUF
                pltpu.async_copy(table.at[_idx(src_v, j + LOOKAHEAD)],
                                 bufs[bg], gs[bg])

            pltpu.make_async_copy(table.at[_idx(src_v, j)], bufs[k], gs[k]).wait()
            pltpu.async_copy(bufs[k], acc.at[_idx(dst_v, j)], ss[k], add=True)

    pltpu.sync_copy(table.at[src_v.at[pl.ds(CPT * CHUNK, TAIL)]], tbuf)
    pltpu.sync_copy(tbuf, acc.at[dst_v.at[pl.ds(CPT * CHUNK, TAIL)]], add=True)

    for j in range(CPT - 2, CPT):
        pltpu.make_async_copy(
            bufs[j % NBUF], acc.at[_idx(dst_v, j)], ss[j % NBUF]).wait()

    plsc.subcore_barrier()
    pltpu.sync_copy(
        acc.at[pl.ds(s * ROWS_PER_TILE, ROWS_PER_TILE)],
        out.at[c, pl.ds(s * ROWS_PER_TILE, ROWS_PER_TILE)],
    )


def _make_deg():
    return pl.kernel(
        _deg_kernel,
        out_type=jax.ShapeDtypeStruct((2, NPAD, DEGW), jnp.float32),
        mesh=_sc_mesh(),
        compiler_params=pltpu.CompilerParams(use_tc_tiling_on_sc=False),
        scratch_types=[
            pltpu.VMEM((EPW,), jnp.int32),
            pltpu.VMEM((CHUNK, DEGW), jnp.float32),
            pltpu.VMEM_SHARED((NPAD, DEGW), jnp.float32),
        ] + [pltpu.SemaphoreType.DMA] * 6,
    )


def _make_agg(H):
    return pl.kernel(
        functools.partial(_agg_body, H),
        out_type=jax.ShapeDtypeStruct((2, NPAD, H), jnp.float32),
        mesh=_sc_mesh(),
        compiler_params=pltpu.CompilerParams(use_tc_tiling_on_sc=False),
        scratch_types=[
            pltpu.VMEM((EPW,), jnp.int32),
            pltpu.VMEM((EPW,), jnp.int32),
            pltpu.VMEM((TAIL, H), jnp.float32),
        ] + [pltpu.VMEM((CHUNK, H), jnp.float32)] * NBUF
        + [pltpu.VMEM_SHARED((NPAD, H), jnp.float32)]
        + [pltpu.SemaphoreType.DMA] * (2 * NBUF),
    )


QR = NPAD // 4
QBLK = 256
GRID = QR // QBLK


def _dinv_quad(degp_ref, width):
    rows = degp_ref.shape[1]
    lane = lax.broadcasted_iota(jnp.int32, (rows, 4 * width), 1) // width
    out = jnp.zeros((rows, 4 * width), jnp.float32)
    for q in range(4):
        d = lax.rsqrt(degp_ref[0, :, DEGW * q:DEGW * q + 1]
                      + degp_ref[1, :, DEGW * q:DEGW * q + 1] + 1.0)
        out = jnp.where(lane == q, d, out)
    return out


def _slice_body(ei_ref, s_ref, d_ref):
    s_ref[...] = ei_ref[0, :]
    d_ref[...] = ei_ref[1, :]


def _prep1_body(x4_ref, w1q_ref, degp_ref, o_ref):
    h1 = jnp.dot(x4_ref[...], w1q_ref[...], preferred_element_type=jnp.float32)
    o_ref[...] = h1 * _dinv_quad(degp_ref, H1)


def _mid_body(aggp_ref, h1p_ref, degp_ref, w2q_ref, b1q_ref, o_ref):
    agg = aggp_ref[0] + aggp_ref[1]
    z1 = jnp.maximum(
        _dinv_quad(degp_ref, H1) * (agg + h1p_ref[...]) + b1q_ref[...], 0.0)
    h2 = jnp.dot(z1, w2q_ref[...], preferred_element_type=jnp.float32)
    o_ref[...] = h2 * _dinv_quad(degp_ref, H2)


def _fin_body(aggp_ref, h2p_ref, degp_ref, wcq_ref, b2q_ref, bc_ref, o_ref):
    agg = aggp_ref[0] + aggp_ref[1]
    z2 = jnp.maximum(
        _dinv_quad(degp_ref, H2) * (agg + h2p_ref[...]) + b2q_ref[...], 0.0)
    logit = jnp.dot(z2, wcq_ref[...], preferred_element_type=jnp.float32)
    o_ref[...] = jax.nn.sigmoid(logit + bc_ref[...])


def _row_spec(rows, last):
    return pl.BlockSpec((rows, last), lambda i: (i, 0))


def _part_spec(rows, last):
    return pl.BlockSpec((2, rows, last), lambda i: (0, i, 0))


def _full_spec(shape):
    return pl.BlockSpec(shape, lambda i: tuple(0 for _ in shape))


def _bdiag4(w):
    return jax.scipy.linalg.block_diag(w, w, w, w)


def kernel(x, edge_index, W1, b1, W2, b2, Wc, bc):
    src1d, dst1d = pl.pallas_call(
        _slice_body,
        grid=(1,),
        in_specs=[pl.BlockSpec((2, E), lambda i: (0, 0))],
        out_specs=[pl.BlockSpec((E,), lambda i: (0,))] * 2,
        out_shape=[jax.ShapeDtypeStruct((E,), jnp.int32)] * 2,
    )(edge_index)

    x4 = jnp.pad(x, ((0, NPAD - N), (0, 0))).reshape(QR, 4 * D)
    w1q = _bdiag4(W1)
    w2q = _bdiag4(W2)
    wcq = _bdiag4(Wc)
    b1q = jnp.tile(b1, 4).reshape(1, 4 * H1)
    b2q = jnp.tile(b2, 4).reshape(1, 4 * H2)

    zeros8 = jnp.zeros((ROWS_PER_TILE, DEGW), jnp.float32)
    ones8 = jnp.ones((CHUNK, DEGW), jnp.float32)
    degp = _make_deg()(dst1d, zeros8, ones8)
    degp4 = degp.reshape(2, QR, 4 * DEGW)

    h1p = pl.pallas_call(
        _prep1_body,
        grid=(GRID,),
        in_specs=[_row_spec(QBLK, 4 * D), _full_spec((4 * D, 4 * H1)),
                  _part_spec(QBLK, 4 * DEGW)],
        out_specs=_row_spec(QBLK, 4 * H1),
        out_shape=jax.ShapeDtypeStruct((QR, 4 * H1), jnp.float32),
    )(x4, w1q, degp4)

    agg1 = _make_agg(H1)(h1p.reshape(NPAD, H1), src1d, dst1d)

    h2p = pl.pallas_call(
        _mid_body,
        grid=(GRID,),
        in_specs=[
            _part_spec(QBLK, 4 * H1),
            _row_spec(QBLK, 4 * H1),
            _part_spec(QBLK, 4 * DEGW),
            _full_spec((4 * H1, 4 * H2)),
            _full_spec((1, 4 * H1)),
        ],
        out_specs=_row_spec(QBLK, 4 * H2),
        out_shape=jax.ShapeDtypeStruct((QR, 4 * H2), jnp.float32),
    )(agg1.reshape(2, QR, 4 * H1), h1p, degp4, w2q, b1q)

    agg2 = _make_agg(H2)(h2p.reshape(NPAD, H2), src1d, dst1d)

    out4 = pl.pallas_call(
        _fin_body,
        grid=(GRID,),
        in_specs=[
            _part_spec(QBLK, 4 * H2),
            _row_spec(QBLK, 4 * H2),
            _part_spec(QBLK, 4 * DEGW),
            _full_spec((4 * H2, 4)),
            _full_spec((1, 4 * H2)),
            _full_spec((1, 1)),
        ],
        out_specs=_row_spec(QBLK, 4),
        out_shape=jax.ShapeDtypeStruct((QR, 4), jnp.float32),
    )(agg2.reshape(2, QR, 4 * H2), h2p, degp4, wcq, b2q, bc.reshape(1, 1))

    return out4.reshape(NPAD, 1)[:N]


# speedup vs baseline: 1.0874x; 1.0874x over previous
import functools

import jax
import jax.numpy as jnp
from jax import lax
from jax.experimental import pallas as pl
from jax.experimental.pallas import tpu as pltpu
from jax.experimental.pallas import tpu_sc as plsc

N = 10000
D = 128
H1 = 64
H2 = 32
E = 320000

NPAD = 10240
ROWS_PER_TILE = NPAD // 16
EPW = E // 32
CHUNK = 128
CPT = 78
TAIL = EPW - CPT * CHUNK
DEGW = 8


def _sc_mesh():
    return plsc.VectorSubcoreMesh(core_axis_name="c", subcore_axis_name="s")


def _idx(ref, j):
    return ref.at[pl.ds(j * CHUNK, CHUNK)]


def _zero_rows(buf, nrows, ncols):
    zv = jnp.zeros((16,), jnp.float32)

    def body(i, _):
        for j in range(ncols // 16):
            buf[i, pl.ds(j * 16, 16)] = zv
        return 0

    lax.fori_loop(0, nrows, body, 0)


def _deg_kernel(dst1d, zeros8, ones8, out, dst_v, ones_v, acc, *ss):
    c = lax.axis_index("c")
    s = lax.axis_index("s")
    wid = c * 16 + s

    pltpu.sync_copy(ones8, ones_v)
    pltpu.sync_copy(zeros8, acc.at[pl.ds(s * ROWS_PER_TILE, ROWS_PER_TILE)])
    pltpu.sync_copy(dst1d.at[pl.ds(wid * EPW, EPW)], dst_v)
    plsc.subcore_barrier()

    @pl.loop(0, CPT, step=6)
    def _(j0):
        for k in range(6):
            j = j0 + k

            @pl.when(j >= 6)
            def _():
                pltpu.make_async_copy(
                    ones_v, acc.at[_idx(dst_v, j - 6)], ss[k]).wait()

            pltpu.async_copy(ones_v, acc.at[_idx(dst_v, j)], ss[k], add=True)

    pltpu.sync_copy(ones_v.at[pl.ds(0, TAIL)],
                    acc.at[dst_v.at[pl.ds(CPT * CHUNK, TAIL)]], add=True)

    for j in range(CPT - 6, CPT):
        pltpu.make_async_copy(
            ones_v, acc.at[_idx(dst_v, j)], ss[j % 6]).wait()

    plsc.subcore_barrier()
    pltpu.sync_copy(
        acc.at[pl.ds(s * ROWS_PER_TILE, ROWS_PER_TILE)],
        out.at[c, pl.ds(s * ROWS_PER_TILE, ROWS_PER_TILE)],
    )


NBUF = 6
LOOKAHEAD = 4


def _agg_body(H, table, src1d, dst1d, out, src_v, dst_v, tbuf, *rest):
    bufs = rest[:NBUF]
    acc = rest[NBUF]
    gs = rest[NBUF + 1:NBUF + 1 + NBUF]
    ss = rest[NBUF + 1 + NBUF:]
    c = lax.axis_index("c")
    s = lax.axis_index("s")
    wid = c * 16 + s
    row0 = s * ROWS_PER_TILE

    _zero_rows(bufs[0], CHUNK, H)
    for r in range(ROWS_PER_TILE // CHUNK):
        pltpu.sync_copy(bufs[0], acc.at[pl.ds(row0 + r * CHUNK, CHUNK)])

    pltpu.sync_copy(src1d.at[pl.ds(wid * EPW, EPW)], src_v)
    pltpu.sync_copy(dst1d.at[pl.ds(wid * EPW, EPW)], dst_v)

    for k in range(LOOKAHEAD):
        pltpu.async_copy(table.at[_idx(src_v, k)], bufs[k], gs[k])
    plsc.subcore_barrier()

    @pl.loop(0, CPT, step=NBUF)
    def _(j0):
        for k in range(NBUF):
            j = j0 + k
            bw = (k + NBUF - 2) % NBUF

            @pl.when(j >= 2)
            def _():
                pltpu.make_async_copy(
                    bufs[bw], acc.at[_idx(dst_v, j - 2)], ss[bw]).wait()

            @pl.when(j + LOOKAHEAD < CPT)
            def _():
                bg = (k + LOOKAHEAD) % NBUF
                pltpu.async_copy(table.at[_idx(src_v, j + LOOKAHEAD)],
                                 bufs[bg], gs[bg])

            pltpu.make_async_copy(table.at[_idx(src_v, j)], bufs[k], gs[k]).wait()
            pltpu.async_copy(bufs[k], acc.at[_idx(dst_v, j)], ss[k], add=True)

    pltpu.sync_copy(table.at[src_v.at[pl.ds(CPT * CHUNK, TAIL)]], tbuf)
    pltpu.sync_copy(tbuf, acc.at[dst_v.at[pl.ds(CPT * CHUNK, TAIL)]], add=True)

    for j in range(CPT - 2, CPT):
        pltpu.make_async_copy(
            bufs[j % NBUF], acc.at[_idx(dst_v, j)], ss[j % NBUF]).wait()

    plsc.subcore_barrier()
    pltpu.sync_copy(
        acc.at[pl.ds(s * ROWS_PER_TILE, ROWS_PER_TILE)],
        out.at[c, pl.ds(s * ROWS_PER_TILE, ROWS_PER_TILE)],
    )


def _make_deg():
    return pl.kernel(
        _deg_kernel,
        out_type=jax.ShapeDtypeStruct((2, NPAD, DEGW), jnp.float32),
        mesh=_sc_mesh(),
        compiler_params=pltpu.CompilerParams(use_tc_tiling_on_sc=False),
        scratch_types=[
            pltpu.VMEM((EPW,), jnp.int32),
            pltpu.VMEM((CHUNK, DEGW), jnp.float32),
            pltpu.VMEM_SHARED((NPAD, DEGW), jnp.float32),
        ] + [pltpu.SemaphoreType.DMA] * 6,
    )


def _make_agg(H):
    return pl.kernel(
        functools.partial(_agg_body, H),
        out_type=jax.ShapeDtypeStruct((2, NPAD, H), jnp.float32),
        mesh=_sc_mesh(),
        compiler_params=pltpu.CompilerParams(use_tc_tiling_on_sc=False),
        scratch_types=[
            pltpu.VMEM((EPW,), jnp.int32),
            pltpu.VMEM((EPW,), jnp.int32),
            pltpu.VMEM((TAIL, H), jnp.float32),
        ] + [pltpu.VMEM((CHUNK, H), jnp.float32)] * NBUF
        + [pltpu.VMEM_SHARED((NPAD, H), jnp.float32)]
        + [pltpu.SemaphoreType.DMA] * (2 * NBUF),
    )


QR = NPAD // 4
QBLK = 256
GRID = QR // QBLK


def _dinv_quad(degp_ref, width):
    rows = degp_ref.shape[1]
    lane = lax.broadcasted_iota(jnp.int32, (rows, 4 * width), 1) // width
    out = jnp.zeros((rows, 4 * width), jnp.float32)
    for q in range(4):
        d = lax.rsqrt(degp_ref[0, :, DEGW * q:DEGW * q + 1]
                      + degp_ref[1, :, DEGW * q:DEGW * q + 1] + 1.0)
        out = jnp.where(lane == q, d, out)
    return out


def _slice_body(ei_ref, s_ref, d_ref):
    s_ref[...] = ei_ref[0, :]
    d_ref[...] = ei_ref[1, :]


def _prep1_body(x4_ref, w1q_ref, degp_ref, o_ref):
    h1 = jnp.dot(x4_ref[...], w1q_ref[...], preferred_element_type=jnp.float32)
    o_ref[...] = h1 * _dinv_quad(degp_ref, H1)


def _mid_body(aggp_ref, h1p_ref, degp_ref, w2q_ref, b1q_ref, o_ref):
    agg = aggp_ref[0] + aggp_ref[1]
    z1 = jnp.maximum(
        _dinv_quad(degp_ref, H1) * (agg + h1p_ref[...]) + b1q_ref[...], 0.0)
    h2 = jnp.dot(z1, w2q_ref[...], preferred_element_type=jnp.float32)
    o_ref[...] = h2 * _dinv_quad(degp_ref, H2)


def _fin_body(aggp_ref, h2p_ref, degp_ref, wcq_ref, b2q_ref, bc_ref, o_ref):
    agg = aggp_ref[0] + aggp_ref[1]
    z2 = jnp.maximum(
        _dinv_quad(degp_ref, H2) * (agg + h2p_ref[...]) + b2q_ref[...], 0.0)
    logit = jnp.dot(z2, wcq_ref[...], preferred_element_type=jnp.float32)
    o_ref[...] = jax.nn.sigmoid(logit + bc_ref[...])


def _row_spec(rows, last):
    return pl.BlockSpec((rows, last), lambda i: (i, 0))


def _part_spec(rows, last):
    return pl.BlockSpec((2, rows, last), lambda i: (0, i, 0))


def _full_spec(shape):
    return pl.BlockSpec(shape, lambda i: tuple(0 for _ in shape))


def _bdiag4(w):
    return jax.scipy.linalg.block_diag(w, w, w, w)


def kernel(x, edge_index, W1, b1, W2, b2, Wc, bc):
    src1d, dst1d = pl.pallas_call(
        _slice_body,
        grid=(1,),
        in_specs=[pl.BlockSpec((2, E), lambda i: (0, 0))],
        out_specs=[pl.BlockSpec((E,), lambda i: (0,))] * 2,
        out_shape=[jax.ShapeDtypeStruct((E,), jnp.int32)] * 2,
    )(edge_index)

    x4 = jnp.pad(x, ((0, NPAD - N), (0, 0))).reshape(QR, 4 * D)
    w1q = _bdiag4(W1)
    w2q = _bdiag4(W2)
    wcq = _bdiag4(Wc)
    b1q = jnp.tile(b1, 4).reshape(1, 4 * H1)
    b2q = jnp.tile(b2, 4).reshape(1, 4 * H2)

    zeros8 = jnp.zeros((ROWS_PER_TILE, DEGW), jnp.float32)
    ones8 = jnp.ones((CHUNK, DEGW), jnp.float32)
    degp = _make_deg()(dst1d, zeros8, ones8)
    degp4 = degp.reshape(2, QR, 4 * DEGW)

    h1p = pl.pallas_call(
        _prep1_body,
        grid=(GRID,),
        in_specs=[_row_spec(QBLK, 4 * D), _full_spec((4 * D, 4 * H1)),
                  _part_spec(QBLK, 4 * DEGW)],
        out_specs=_row_spec(QBLK, 4 * H1),
        out_shape=jax.ShapeDtypeStruct((QR, 4 * H1), jnp.float32),
    )(x4, w1q, degp4)

    agg1 = _make_agg(H1)(h1p.reshape(NPAD, H1), src1d, dst1d)

    h2p = pl.pallas_call(
        _mid_body,
        grid=(GRID,),
        in_specs=[
            _part_spec(QBLK, 4 * H1),
            _row_spec(QBLK, 4 * H1),
            _part_spec(QBLK, 4 * DEGW),
            _full_spec((4 * H1, 4 * H2)),
            _full_spec((1, 4 * H1)),
        ],
        out_specs=_row_spec(QBLK, 4 * H2),
        out_shape=jax.ShapeDtypeStruct((QR, 4 * H2), jnp.float32),
    )(agg1.reshape(2, QR, 4 * H1), h1p, degp4, w2q, b1q)

    agg2 = _make_agg(H2)(h2p.reshape(NPAD, H2), src1d, dst1d)

    out4 = pl.pallas_call(
        _fin_body,
        grid=(GRID,),
        in_specs=[
            _part_spec(QBLK, 4 * H2),
            _row_spec(QBLK, 4 * H2),
            _part_spec(QBLK, 4 * DEGW),
            _full_spec((4 * H2, 4)),
            _full_spec((1, 4 * H2)),
            _full_spec((1, 1)),
        ],
        out_specs=_row_spec(QBLK, 4),
        out_shape=jax.ShapeDtypeStruct((QR, 4), jnp.float32),
    )(agg2.reshape(2, QR, 4 * H2), h2p, degp4, wcq, b2q, bc.reshape(1, 1))

    return out4.reshape(NPAD, 1)[:N]
